# Initial kernel scaffold; baseline (speedup 1.0000x reference)
import jax
import jax.numpy as jnp
from jax.experimental import pallas as pl


def kernel(pred_scores, pred_rboxes, anchor_points, gt_labels, gt_bboxes, gt_poses, gt_crowd, pad_gt_mask, bg_index):
    raise NotImplementedError("write your pallas kernel here")


# speedup vs baseline: 20.8215x; 20.8215x over previous
import jax
import jax.numpy as jnp
from jax import lax
from jax.experimental import pallas as pl
from jax.experimental.pallas import tpu as pltpu

TOPK = 13
EPS = 1e-09
IOU_EPS = 1e-09


def _rot_minmax(cx, cy, w, h, r):
    cos_r = jnp.cos(r)
    sin_r = jnp.sin(r)
    dx = w * 0.5 * cos_r
    dy = h * 0.5 * sin_r
    xm, xp = cx - dx, cx + dx
    ym, yp = cy - dy, cy + dy
    xs = (xm, xp, xp, xm)
    ys = (ym, ym, yp, yp)
    xr = [cx + (xc - cx) * cos_r - (yc - cy) * sin_r for xc, yc in zip(xs, ys)]
    yr = [cy + (xc - cx) * sin_r + (yc - cy) * cos_r for xc, yc in zip(xs, ys)]
    minx = jnp.minimum(jnp.minimum(xr[0], xr[1]), jnp.minimum(xr[2], xr[3]))
    maxx = jnp.maximum(jnp.maximum(xr[0], xr[1]), jnp.maximum(xr[2], xr[3]))
    miny = jnp.minimum(jnp.minimum(yr[0], yr[1]), jnp.minimum(yr[2], yr[3]))
    maxy = jnp.maximum(jnp.maximum(yr[0], yr[1]), jnp.maximum(yr[2], yr[3]))
    return minx, maxx, miny, maxy


def _body(ps_ref, rb_ref, ap_ref, gtl_ref, gtb_ref, gtc_ref, pad_ref, bg_ref,
          out_ref):
    f32 = jnp.float32
    psT = ps_ref[0]
    rbT = rb_ref[0]
    apT = ap_ref[...]
    gtb = gtb_ref[0]
    gtl = gtl_ref[0]
    gtc = gtc_ref[0]
    pad = pad_ref[0]
    C, L = psT.shape
    n = gtb.shape[0]

    pcx, pcy = rbT[0:1, :], rbT[1:2, :]
    pw, ph, pr = rbT[2:3, :], rbT[3:4, :], rbT[4:5, :]
    px, py = apT[0:1, :], apT[1:2, :]

    gcx, gcy = gtb[:, 0:1], gtb[:, 1:2]
    gw, gh, gr = gtb[:, 2:3], gtb[:, 3:4], gtb[:, 4:5]

    g_minx, g_maxx, g_miny, g_maxy = _rot_minmax(gcx, gcy, gw, gh, gr)
    p_minx, p_maxx, p_miny, p_maxy = _rot_minmax(pcx, pcy, pw, ph, pr)

    iw = jnp.clip(jnp.minimum(g_maxx, p_maxx) - jnp.maximum(g_minx, p_minx),
                  0.0, None)
    ih = jnp.clip(jnp.minimum(g_maxy, p_maxy) - jnp.maximum(g_miny, p_miny),
                  0.0, None)
    inter = iw * ih
    union = gw * gh + pw * ph - inter
    iou = jnp.clip(inter / (union + IOU_EPS), 0.0, 1.0)

    cls_oh = (gtl == lax.broadcasted_iota(jnp.int32, (n, C), 1)).astype(f32)
    cls = lax.dot_general(cls_oh, psT, (((1,), (0,)), ((), ())),
                          precision=lax.Precision.HIGHEST,
                          preferred_element_type=f32)

    iou2 = iou * iou
    iou4 = iou2 * iou2
    metric = cls * (iou4 * iou2)

    cosg = jnp.cos(gr)
    sing = jnp.sin(gr)
    dxm = px - gcx
    dym = py - gcy
    lx = dxm * cosg + dym * sing
    ly = -dxm * sing + dym * cosg
    inside = ((jnp.abs(lx) <= gw * 0.5) & (jnp.abs(ly) <= gh * 0.5)).astype(f32)

    iota_l = lax.broadcasted_iota(jnp.int32, (n, L), 1)
    cur = metric * inside
    tk = jnp.zeros((n, L), f32)
    for _ in range(TOPK):
        m = jnp.max(cur, axis=1, keepdims=True)
        idx = jnp.min(jnp.where(cur == m, iota_l, L), axis=1,
                      keepdims=True)
        sel = iota_l == idx
        tk = jnp.where(sel, 1.0, tk)
        cur = jnp.where(sel, -1.0, cur)

    mask_pos = tk * pad * inside
    mps = jnp.sum(mask_pos, axis=0, keepdims=True)
    multiple = mps > 1.0

    iota_g = lax.broadcasted_iota(jnp.int32, (n, L), 0)
    mg = jnp.max(iou, axis=0, keepdims=True)
    gidx = jnp.min(jnp.where(iou == mg, iota_g, n), axis=0, keepdims=True)
    onehot_max = (iota_g == gidx).astype(f32)
    fm = jnp.where(multiple, onehot_max, mask_pos)

    any_pos = jnp.sum(fm, axis=0, keepdims=True) > 0.0
    agi = jnp.sum(fm * iota_g.astype(f32), axis=0, keepdims=True)

    amr = metric * fm
    mm = jnp.max(amr, axis=1, keepdims=True)
    mi = jnp.max(iou * fm, axis=1, keepdims=True)
    amn = amr / (mm + EPS) * mi
    amv = jnp.max(amn, axis=0, keepdims=True)

    bgf = bg_ref[0, 0].astype(f32)
    lab = jnp.where(any_pos, jnp.sum(fm * gtl.astype(f32), axis=0,
                                     keepdims=True), bgf)
    crwf = gtc.astype(f32)
    crw = jnp.where(any_pos, jnp.sum(fm * crwf, axis=0, keepdims=True),
                    crwf[0, 0])

    iota_c = lax.broadcasted_iota(jnp.int32, (C, L), 0).astype(f32)
    out_ref[0, 0:C, :] = jnp.where(iota_c == lab, amv, 0.0)
    out_ref[0, C:C + 1, :] = lab
    for c in range(5):
        out_ref[0, C + 1 + c:C + 2 + c, :] = jnp.where(
            any_pos, jnp.sum(fm * gtb[:, c:c + 1], axis=0, keepdims=True),
            gtb[0, c])
    out_ref[0, C + 6:C + 7, :] = agi
    out_ref[0, C + 7:C + 8, :] = crw


def kernel(pred_scores, pred_rboxes, anchor_points, gt_labels, gt_bboxes,
           gt_poses, gt_crowd, pad_gt_mask, bg_index):
    B, L, C = pred_scores.shape
    n = gt_bboxes.shape[1]
    R = C + 9
    bg_arr = jnp.reshape(jnp.asarray(bg_index, jnp.int32), (1, 1))
    psT = jnp.transpose(pred_scores, (0, 2, 1))
    rbT = jnp.transpose(pred_rboxes, (0, 2, 1))
    apT = jnp.transpose(anchor_points, (1, 0))

    packed = pl.pallas_call(
        _body,
        grid=(B,),
        in_specs=[
            pl.BlockSpec((1, C, L), lambda b: (b, 0, 0)),
            pl.BlockSpec((1, 5, L), lambda b: (b, 0, 0)),
            pl.BlockSpec((2, L), lambda b: (0, 0)),
            pl.BlockSpec((1, n, 1), lambda b: (b, 0, 0)),
            pl.BlockSpec((1, n, 5), lambda b: (b, 0, 0)),
            pl.BlockSpec((1, n, 1), lambda b: (b, 0, 0)),
            pl.BlockSpec((1, n, 1), lambda b: (b, 0, 0)),
            pl.BlockSpec((1, 1), lambda b: (0, 0)),
        ],
        out_specs=pl.BlockSpec((1, R, L), lambda b: (b, 0, 0)),
        out_shape=jax.ShapeDtypeStruct((B, R, L), jnp.float32),
    )(psT, rbT, apT, gt_labels, gt_bboxes, gt_crowd, pad_gt_mask, bg_arr)

    scores = jnp.transpose(packed[:, 0:C, :], (0, 2, 1))
    labels = packed[:, C, :].astype(jnp.int32)
    rboxes = jnp.transpose(packed[:, C + 1:C + 6, :], (0, 2, 1))
    agi = packed[:, C + 6, :].astype(jnp.int32)
    crowd = packed[:, C + 7, :].astype(bool)
    return labels, rboxes, scores, agi, crowd
